# Initial kernel scaffold; baseline (speedup 1.0000x reference)
import functools

import jax
import jax.numpy as jnp
from jax import lax
from jax.experimental import pallas as pl
from jax.experimental.pallas import tpu as pltpu
from jax.experimental.pallas import tpu_sc as plsc

N = 10000
NP = 10240
D = 128
E = 320000
NUM_TILES = 32
EDGES_PER_TILE = E // NUM_TILES
CHUNK = 125
CHUNKS_PER_TILE = EDGES_PER_TILE // CHUNK
ROWS_PER_BLOCK = 1024
GRID = NP // ROWS_PER_BLOCK


_MESH = plsc.VectorSubcoreMesh(core_axis_name="c", subcore_axis_name="s")


@functools.partial(
    pl.kernel,
    out_type=jax.ShapeDtypeStruct((2, NP), jnp.float32),
    mesh=_MESH,
    scratch_types=[
        pltpu.VMEM_SHARED((NP,), jnp.float32),
        pltpu.VMEM((CHUNKS_PER_TILE, CHUNK), jnp.int32),
        pltpu.VMEM((CHUNK,), jnp.float32),
    ],
)
def _sc_degree(dst_hbm, ones_hbm, zeros_hbm, out_hbm, deg_sh, dst_v, ones_v):
    c = lax.axis_index("c")
    s = lax.axis_index("s")
    wid = c * 16 + s

    @pl.when(s == 0)
    def _zero():
        pltpu.sync_copy(zeros_hbm, deg_sh)

    pltpu.sync_copy(dst_hbm.at[pl.ds(wid * CHUNKS_PER_TILE, CHUNKS_PER_TILE)],
                    dst_v)
    pltpu.sync_copy(ones_hbm, ones_v)
    plsc.subcore_barrier()

    def body(j, carry):
        pltpu.sync_copy(ones_v, deg_sh.at[dst_v.at[j]], add=True)
        return carry

    lax.fori_loop(0, CHUNKS_PER_TILE, body, 0)
    plsc.subcore_barrier()

    @pl.when(s == 0)
    def _flush():
        pltpu.sync_copy(deg_sh, out_hbm.at[c])


@functools.partial(
    pl.kernel,
    out_type=jax.ShapeDtypeStruct((2, NP, D), jnp.float32),
    mesh=_MESH,
    scratch_types=[
        pltpu.VMEM_SHARED((NP, D), jnp.float32),
        pltpu.VMEM((CHUNKS_PER_TILE, CHUNK), jnp.int32),
        pltpu.VMEM((CHUNKS_PER_TILE, CHUNK), jnp.int32),
        pltpu.VMEM((CHUNK, D), jnp.float32),
        pltpu.VMEM((CHUNK, D), jnp.float32),
        pltpu.SemaphoreType.DMA,
        pltpu.SemaphoreType.DMA,
    ],
)
def _sc_scatter(hp_hbm, src_hbm, dst_hbm, zeros_hbm, out_hbm,
                acc_sh, src_v, dst_v, buf_a, buf_b, sem_a, sem_b):
    c = lax.axis_index("c")
    s = lax.axis_index("s")
    wid = c * 16 + s

    @pl.when(s == 0)
    def _zero():
        pltpu.sync_copy(zeros_hbm, acc_sh)

    pltpu.sync_copy(src_hbm.at[pl.ds(wid * CHUNKS_PER_TILE, CHUNKS_PER_TILE)],
                    src_v)
    pltpu.sync_copy(dst_hbm.at[pl.ds(wid * CHUNKS_PER_TILE, CHUNKS_PER_TILE)],
                    dst_v)
    plsc.subcore_barrier()

    pltpu.async_copy(hp_hbm.at[src_v.at[0]], buf_a, sem_a)
    pltpu.async_copy(hp_hbm.at[src_v.at[1]], buf_b, sem_b)

    def body(i, carry):
        ja = 2 * i
        jb = 2 * i + 1
        pltpu.make_async_copy(hp_hbm.at[src_v.at[ja]], buf_a, sem_a).wait()
        pltpu.sync_copy(buf_a, acc_sh.at[dst_v.at[ja]], add=True)

        @pl.when(ja + 2 < CHUNKS_PER_TILE)
        def _next_a():
            pltpu.async_copy(hp_hbm.at[src_v.at[ja + 2]], buf_a, sem_a)

        pltpu.make_async_copy(hp_hbm.at[src_v.at[jb]], buf_b, sem_b).wait()
        pltpu.sync_copy(buf_b, acc_sh.at[dst_v.at[jb]], add=True)

        @pl.when(jb + 2 < CHUNKS_PER_TILE)
        def _next_b():
            pltpu.async_copy(hp_hbm.at[src_v.at[jb + 2]], buf_b, sem_b)

        return carry

    lax.fori_loop(0, CHUNKS_PER_TILE // 2, body, 0)
    plsc.subcore_barrier()

    @pl.when(s == 0)
    def _flush():
        pltpu.sync_copy(acc_sh, out_hbm.at[c])


def _dinv_col(deg_blk):
    deg = deg_blk[0] + deg_blk[1] + 1.0
    dinv = lax.rsqrt(deg)
    r_sub = lax.broadcasted_iota(jnp.int32, (ROWS_PER_BLOCK, 8), 0) // 128
    k_sub = lax.broadcasted_iota(jnp.int32, (ROWS_PER_BLOCK, 8), 1)
    onehot = (r_sub == k_sub).astype(jnp.float32)
    rows = jnp.dot(onehot, dinv, preferred_element_type=jnp.float32)
    r_lane = lax.broadcasted_iota(jnp.int32, (ROWS_PER_BLOCK, 128), 0) % 128
    m_lane = lax.broadcasted_iota(jnp.int32, (ROWS_PER_BLOCK, 128), 1)
    sel = (r_lane == m_lane).astype(jnp.float32)
    return jnp.sum(rows * sel, axis=1, keepdims=True)


def _tc1_body(x_ref, w_ref, deg_ref, hp_ref, st_ref):
    h = jnp.dot(x_ref[...], w_ref[...], preferred_element_type=jnp.float32)
    dinv = _dinv_col(deg_ref[...])
    hp_ref[...] = h * dinv
    st_ref[...] = h * (dinv * dinv)


def _tc2_body(acc_ref, st_ref, deg_ref, w_ref, b_ref, hp_ref, st2_ref):
    dinv = _dinv_col(deg_ref[...])
    acc = acc_ref[0] + acc_ref[1]
    out1 = acc * dinv + st_ref[...] + b_ref[...]
    h2 = jnp.dot(out1, w_ref[...], preferred_element_type=jnp.float32)
    hp_ref[...] = h2 * dinv
    st2_ref[...] = h2 * (dinv * dinv)


def _tc3_body(acc_ref, st_ref, deg_ref, b_ref, out_ref):
    dinv = _dinv_col(deg_ref[...])
    acc = acc_ref[0] + acc_ref[1]
    out_ref[...] = acc * dinv + st_ref[...] + b_ref[...]


_row_spec = pl.BlockSpec((ROWS_PER_BLOCK, D), lambda i: (i, 0))
_w_spec = pl.BlockSpec((D, D), lambda i: (0, 0))
_b_spec = pl.BlockSpec((1, D), lambda i: (0, 0))
_deg_spec = pl.BlockSpec((2, 8, D), lambda i: (0, i, 0))
_acc_spec = pl.BlockSpec((2, ROWS_PER_BLOCK, D), lambda i: (0, i, 0))

_tc1 = pl.pallas_call(
    _tc1_body,
    grid=(GRID,),
    in_specs=[_row_spec, _w_spec, _deg_spec],
    out_specs=[_row_spec, _row_spec],
    out_shape=[jax.ShapeDtypeStruct((NP, D), jnp.float32)] * 2,
)

_tc2 = pl.pallas_call(
    _tc2_body,
    grid=(GRID,),
    in_specs=[_acc_spec, _row_spec, _deg_spec, _w_spec, _b_spec],
    out_specs=[_row_spec, _row_spec],
    out_shape=[jax.ShapeDtypeStruct((NP, D), jnp.float32)] * 2,
)

_tc3 = pl.pallas_call(
    _tc3_body,
    grid=(GRID,),
    in_specs=[_acc_spec, _row_spec, _deg_spec, _b_spec],
    out_specs=_row_spec,
    out_shape=jax.ShapeDtypeStruct((NP, D), jnp.float32),
)


def kernel(x, edge_index, W1, b1, W2, b2):
    src = edge_index[0].astype(jnp.int32).reshape(NUM_TILES * CHUNKS_PER_TILE,
                                                  CHUNK)
    dst = edge_index[1].astype(jnp.int32).reshape(NUM_TILES * CHUNKS_PER_TILE,
                                                  CHUNK)
    x_p = jnp.pad(x, ((0, NP - N), (0, 0)))
    ones_c = jnp.ones((CHUNK,), jnp.float32)
    zeros_n = jnp.zeros((NP,), jnp.float32)
    zeros_nd = jnp.zeros((NP, D), jnp.float32)
    b1r = b1.reshape(1, D)
    b2r = b2.reshape(1, D)

    deg2 = _sc_degree(dst, ones_c, zeros_n).reshape(2, NP // 128, 128)

    hp1, st1 = _tc1(x_p, W1, deg2)
    acc1 = _sc_scatter(hp1, src, dst, zeros_nd)
    hp2, st2 = _tc2(acc1, st1, deg2, W2, b1r)
    acc2 = _sc_scatter(hp2, src, dst, zeros_nd)
    out = _tc3(acc2, st2, deg2, b2r)
    return out[:N]


# speedup vs baseline: 32.0417x; 32.0417x over previous
import functools

import jax
import jax.numpy as jnp
from jax import lax
from jax.experimental import pallas as pl
from jax.experimental.pallas import tpu as pltpu
from jax.experimental.pallas import tpu_sc as plsc

N = 10000
NP = 10240
D = 128
E = 320000
NUM_TILES = 32
EDGES_PER_TILE = E // NUM_TILES
CHUNK = 125
CHUNKS_PER_TILE = EDGES_PER_TILE // CHUNK
ROWS_PER_BLOCK = 1024
GRID = NP // ROWS_PER_BLOCK


_MESH = plsc.VectorSubcoreMesh(core_axis_name="c", subcore_axis_name="s")


@functools.partial(
    pl.kernel,
    out_type=jax.ShapeDtypeStruct((2, NP), jnp.float32),
    mesh=_MESH,
    scratch_types=[
        pltpu.VMEM_SHARED((NP,), jnp.float32),
        pltpu.VMEM((CHUNKS_PER_TILE, CHUNK), jnp.int32),
        pltpu.VMEM((CHUNK,), jnp.float32),
    ],
)
def _sc_degree(dst_hbm, ones_hbm, zeros_hbm, out_hbm, deg_sh, dst_v, ones_v):
    c = lax.axis_index("c")
    s = lax.axis_index("s")
    wid = c * 16 + s

    @pl.when(s == 0)
    def _zero():
        pltpu.sync_copy(zeros_hbm, deg_sh)

    pltpu.sync_copy(dst_hbm.at[pl.ds(wid * CHUNKS_PER_TILE, CHUNKS_PER_TILE)],
                    dst_v)
    pltpu.sync_copy(ones_hbm, ones_v)
    plsc.subcore_barrier()

    def body(j, carry):
        pltpu.sync_copy(ones_v, deg_sh.at[dst_v.at[j]], add=True)
        return carry

    lax.fori_loop(0, CHUNKS_PER_TILE, body, 0)
    plsc.subcore_barrier()

    @pl.when(s == 0)
    def _flush():
        pltpu.sync_copy(deg_sh, out_hbm.at[c])


GROUP = 16
NUM_GROUPS = CHUNKS_PER_TILE // GROUP


@functools.partial(
    pl.kernel,
    out_type=jax.ShapeDtypeStruct((2, NP, D), jnp.float32),
    mesh=_MESH,
    scratch_types=[
        pltpu.VMEM_SHARED((NP, D), jnp.float32),
        pltpu.VMEM((2, GROUP, CHUNK), jnp.int32),
        pltpu.VMEM((2, GROUP, CHUNK), jnp.int32),
        pltpu.VMEM((CHUNK, D), jnp.float32),
        pltpu.VMEM((CHUNK, D), jnp.float32),
        pltpu.SemaphoreType.DMA,
        pltpu.SemaphoreType.DMA,
    ],
)
def _sc_scatter(hp_hbm, src_hbm, dst_hbm, zeros_hbm, out_hbm,
                acc_sh, src_v, dst_v, buf_a, buf_b, sem_a, sem_b):
    c = lax.axis_index("c")
    s = lax.axis_index("s")
    wid = c * 16 + s
    base = wid * CHUNKS_PER_TILE

    @pl.when(s == 0)
    def _zero():
        pltpu.sync_copy(zeros_hbm, acc_sh)

    pltpu.sync_copy(src_hbm.at[pl.ds(base, GROUP)], src_v.at[0])
    pltpu.sync_copy(dst_hbm.at[pl.ds(base, GROUP)], dst_v.at[0])
    plsc.subcore_barrier()

    pltpu.async_copy(hp_hbm.at[src_v.at[0, 0]], buf_a, sem_a)
    pltpu.async_copy(hp_hbm.at[src_v.at[0, 1]], buf_b, sem_b)

    def step(j, buf, sem):
        g = j // GROUP
        k = j % GROUP
        slot = g % 2

        @pl.when(jnp.logical_and(k == 0, g + 1 < NUM_GROUPS))
        def _prefetch_group():
            nxt = g + 1
            pltpu.sync_copy(src_hbm.at[pl.ds(base + nxt * GROUP, GROUP)],
                            src_v.at[nxt % 2])
            pltpu.sync_copy(dst_hbm.at[pl.ds(base + nxt * GROUP, GROUP)],
                            dst_v.at[nxt % 2])

        pltpu.make_async_copy(hp_hbm.at[src_v.at[slot, k]], buf, sem).wait()
        pltpu.sync_copy(buf, acc_sh.at[dst_v.at[slot, k]], add=True)

        @pl.when(j + 2 < CHUNKS_PER_TILE)
        def _next():
            jn = j + 2
            pltpu.async_copy(
                hp_hbm.at[src_v.at[(jn // GROUP) % 2, jn % GROUP]], buf, sem)

    def body(i, carry):
        step(2 * i, buf_a, sem_a)
        step(2 * i + 1, buf_b, sem_b)
        return carry

    lax.fori_loop(0, CHUNKS_PER_TILE // 2, body, 0)
    plsc.subcore_barrier()

    @pl.when(s == 0)
    def _flush():
        pltpu.sync_copy(acc_sh, out_hbm.at[c])


def _dinv_col(deg_blk):
    deg = deg_blk[0] + deg_blk[1] + 1.0
    dinv = lax.rsqrt(deg)
    r_sub = lax.broadcasted_iota(jnp.int32, (ROWS_PER_BLOCK, 8), 0) // 128
    k_sub = lax.broadcasted_iota(jnp.int32, (ROWS_PER_BLOCK, 8), 1)
    onehot = (r_sub == k_sub).astype(jnp.float32)
    rows = jnp.dot(onehot, dinv, preferred_element_type=jnp.float32)
    r_lane = lax.broadcasted_iota(jnp.int32, (ROWS_PER_BLOCK, 128), 0) % 128
    m_lane = lax.broadcasted_iota(jnp.int32, (ROWS_PER_BLOCK, 128), 1)
    sel = (r_lane == m_lane).astype(jnp.float32)
    return jnp.sum(rows * sel, axis=1, keepdims=True)


def _tc1_body(x_ref, w_ref, deg_ref, hp_ref, st_ref):
    h = jnp.dot(x_ref[...], w_ref[...], preferred_element_type=jnp.float32)
    dinv = _dinv_col(deg_ref[...])
    hp_ref[...] = h * dinv
    st_ref[...] = h * (dinv * dinv)


def _tc2_body(acc_ref, st_ref, deg_ref, w_ref, b_ref, hp_ref, st2_ref):
    dinv = _dinv_col(deg_ref[...])
    acc = acc_ref[0] + acc_ref[1]
    out1 = acc * dinv + st_ref[...] + b_ref[...]
    h2 = jnp.dot(out1, w_ref[...], preferred_element_type=jnp.float32)
    hp_ref[...] = h2 * dinv
    st2_ref[...] = h2 * (dinv * dinv)


def _tc3_body(acc_ref, st_ref, deg_ref, b_ref, out_ref):
    dinv = _dinv_col(deg_ref[...])
    acc = acc_ref[0] + acc_ref[1]
    out_ref[...] = acc * dinv + st_ref[...] + b_ref[...]


_row_spec = pl.BlockSpec((ROWS_PER_BLOCK, D), lambda i: (i, 0))
_w_spec = pl.BlockSpec((D, D), lambda i: (0, 0))
_b_spec = pl.BlockSpec((1, D), lambda i: (0, 0))
_deg_spec = pl.BlockSpec((2, 8, D), lambda i: (0, i, 0))
_acc_spec = pl.BlockSpec((2, ROWS_PER_BLOCK, D), lambda i: (0, i, 0))

_tc1 = pl.pallas_call(
    _tc1_body,
    grid=(GRID,),
    in_specs=[_row_spec, _w_spec, _deg_spec],
    out_specs=[_row_spec, _row_spec],
    out_shape=[jax.ShapeDtypeStruct((NP, D), jnp.float32)] * 2,
)

_tc2 = pl.pallas_call(
    _tc2_body,
    grid=(GRID,),
    in_specs=[_acc_spec, _row_spec, _deg_spec, _w_spec, _b_spec],
    out_specs=[_row_spec, _row_spec],
    out_shape=[jax.ShapeDtypeStruct((NP, D), jnp.float32)] * 2,
)

_tc3 = pl.pallas_call(
    _tc3_body,
    grid=(GRID,),
    in_specs=[_acc_spec, _row_spec, _deg_spec, _b_spec],
    out_specs=_row_spec,
    out_shape=jax.ShapeDtypeStruct((NP, D), jnp.float32),
)


def kernel(x, edge_index, W1, b1, W2, b2):
    src = edge_index[0].astype(jnp.int32).reshape(NUM_TILES * CHUNKS_PER_TILE,
                                                  CHUNK)
    dst = edge_index[1].astype(jnp.int32).reshape(NUM_TILES * CHUNKS_PER_TILE,
                                                  CHUNK)
    x_p = jnp.pad(x, ((0, NP - N), (0, 0)))
    ones_c = jnp.ones((CHUNK,), jnp.float32)
    zeros_n = jnp.zeros((NP,), jnp.float32)
    zeros_nd = jnp.zeros((NP, D), jnp.float32)
    b1r = b1.reshape(1, D)
    b2r = b2.reshape(1, D)

    deg2 = _sc_degree(dst, ones_c, zeros_n).reshape(2, NP // 128, 128)

    hp1, st1 = _tc1(x_p, W1, deg2)
    acc1 = _sc_scatter(hp1, src, dst, zeros_nd)
    hp2, st2 = _tc2(acc1, st1, deg2, W2, b1r)
    acc2 = _sc_scatter(hp2, src, dst, zeros_nd)
    out = _tc3(acc2, st2, deg2, b2r)
    return out[:N]
